# baseline (speedup 1.0000x reference)
import functools

import jax
import jax.numpy as jnp
from jax import lax
from jax.experimental import pallas as pl
from jax.experimental.pallas import tpu as pltpu
from jax.experimental.pallas import tpu_sc as plsc

_NBUF = 2


def _make_gather(V: int, D: int, BT: int, S: int):
    info = plsc.get_sparse_core_info()
    nc, ns = info.num_cores, info.num_subcores
    nw = nc * ns
    seq_per_w = BT // nw
    assert seq_per_w % _NBUF == 0
    n_outer = seq_per_w // _NBUF

    mesh = plsc.VectorSubcoreMesh(core_axis_name="c", subcore_axis_name="s")

    @functools.partial(
        pl.kernel,
        mesh=mesh,
        out_type=jax.ShapeDtypeStruct((BT, S, D), jnp.float32),
        compiler_params=pltpu.CompilerParams(use_tc_tiling_on_sc=False),
        scratch_types=[
            pltpu.VMEM((_NBUF, S), jnp.int32),
            pltpu.VMEM((_NBUF, S, D), jnp.float32),
            pltpu.SemaphoreType.DMA,
            pltpu.SemaphoreType.DMA,
            pltpu.SemaphoreType.DMA,
            pltpu.SemaphoreType.DMA,
        ],
    )
    def k(table_hbm, x_hbm, out_hbm, idx_v, rows_v, gsem0, gsem1,
          wsem0, wsem1):
        wid = lax.axis_index("s") * nc + lax.axis_index("c")
        b0 = wid * seq_per_w
        gsems = (gsem0, gsem1)
        wsems = (wsem0, wsem1)

        def body(t, carry):
            handles = []
            for u in range(_NBUF):
                b = b0 + t * _NBUF + u

                @pl.when(t > 0)
                def _drain_prev_write(u=u, b=b):
                    pltpu.make_async_copy(
                        rows_v.at[u], out_hbm.at[b - _NBUF],
                        wsems[u]).wait()

                pltpu.sync_copy(x_hbm.at[b], idx_v.at[u])
                handles.append(pltpu.async_copy(
                    table_hbm.at[idx_v.at[u]], rows_v.at[u], gsems[u]))
            for u in range(_NBUF):
                b = b0 + t * _NBUF + u
                handles[u].wait()
                pltpu.async_copy(rows_v.at[u], out_hbm.at[b], wsems[u])
            return carry

        lax.fori_loop(0, n_outer, body, 0)
        for u in range(_NBUF):
            b = b0 + (n_outer - 1) * _NBUF + u
            pltpu.make_async_copy(
                rows_v.at[u], out_hbm.at[b], wsems[u]).wait()

    return k


def kernel(x, table):
    bt, s = x.shape
    v, d = table.shape
    return _make_gather(v, d, bt, s)(table, x)


# speedup vs baseline: 1.0504x; 1.0504x over previous
import functools

import jax
import jax.numpy as jnp
from jax import lax
from jax.experimental import pallas as pl
from jax.experimental.pallas import tpu as pltpu
from jax.experimental.pallas import tpu_sc as plsc

_NBUF = 2
_BLK = 512


def _make_gather(V: int, D: int, BT: int, S: int):
    info = plsc.get_sparse_core_info()
    nc, ns = info.num_cores, info.num_subcores
    nw = nc * ns
    blocks_per_s = BT // _BLK
    units = S * blocks_per_s
    units_per_w = units // nw
    assert units_per_w % _NBUF == 0
    n_outer = units_per_w // _NBUF

    mesh = plsc.VectorSubcoreMesh(core_axis_name="c", subcore_axis_name="s")

    @functools.partial(
        pl.kernel,
        mesh=mesh,
        out_type=jax.ShapeDtypeStruct((S, BT, D), jnp.float32),
        compiler_params=pltpu.CompilerParams(use_tc_tiling_on_sc=False),
        scratch_types=[
            pltpu.VMEM((_NBUF, _BLK), jnp.int32),
            pltpu.VMEM((_NBUF, _BLK, D), jnp.float32),
            pltpu.SemaphoreType.DMA,
            pltpu.SemaphoreType.DMA,
            pltpu.SemaphoreType.DMA,
            pltpu.SemaphoreType.DMA,
        ],
    )
    def k(table_hbm, xt_hbm, out_hbm, idx_v, rows_v, gsem0, gsem1,
          wsem0, wsem1):
        wid = lax.axis_index("s") * nc + lax.axis_index("c")
        u0 = wid * units_per_w
        gsems = (gsem0, gsem1)
        wsems = (wsem0, wsem1)

        def body(t, carry):
            handles = []
            units_now = []
            for u in range(_NBUF):
                unit = u0 + t * _NBUF + u
                s = unit // blocks_per_s
                b0 = (unit % blocks_per_s) * _BLK
                units_now.append((s, b0))

                @pl.when(t > 0)
                def _drain_prev_write(u=u, unit=unit):
                    ps = (unit - _NBUF) // blocks_per_s
                    pb0 = ((unit - _NBUF) % blocks_per_s) * _BLK
                    pltpu.make_async_copy(
                        rows_v.at[u],
                        out_hbm.at[ps, pl.ds(pb0, _BLK)],
                        wsems[u]).wait()

                pltpu.sync_copy(xt_hbm.at[s, pl.ds(b0, _BLK)], idx_v.at[u])
                handles.append(pltpu.async_copy(
                    table_hbm.at[idx_v.at[u]], rows_v.at[u], gsems[u]))
            for u in range(_NBUF):
                s, b0 = units_now[u]
                handles[u].wait()
                pltpu.async_copy(
                    rows_v.at[u], out_hbm.at[s, pl.ds(b0, _BLK)], wsems[u])
            return carry

        lax.fori_loop(0, n_outer, body, 0)
        for u in range(_NBUF):
            unit = u0 + (n_outer - 1) * _NBUF + u
            s = unit // blocks_per_s
            b0 = (unit % blocks_per_s) * _BLK
            pltpu.make_async_copy(
                rows_v.at[u], out_hbm.at[s, pl.ds(b0, _BLK)],
                wsems[u]).wait()

    return k


def kernel(x, table):
    bt, s = x.shape
    v, d = table.shape
    out_t = _make_gather(v, d, bt, s)(table, x.T)
    return out_t.transpose(1, 0, 2)
